# baseline (speedup 1.0000x reference)
import functools

import jax
import jax.numpy as jnp
from jax import lax
from jax.experimental import pallas as pl
from jax.experimental.pallas import tpu as pltpu
from jax.experimental.pallas import tpu_sc as plsc

_N = 10000
_E = 320000
_T = 20
_SEQ_IN = 5
_H = 32
_STATIC = 7
_EMB = 128
_EDGE_F = 5
_MLP_H = 512

_NT = 32
_CH = 128
_NCH = 80
_EPAD = _NT * _NCH * _CH
_NPAD = 10112

_mesh = plsc.VectorSubcoreMesh(
    core_axis_name="c", subcore_axis_name="s", num_cores=2, num_subcores=16)


def _gru_body(seq_ref, st_ref, wir, wiz, win, whr, whz, whn,
              bir, biz, bin_, bhr, bhz, bhn, w1h, w1s, b1, out_ref):
    nb = st_ref.shape[0]

    def step(t, h):
        xs = seq_ref[t]
        i_r = jnp.dot(xs, wir[...], preferred_element_type=jnp.float32) + bir[...]
        i_z = jnp.dot(xs, wiz[...], preferred_element_type=jnp.float32) + biz[...]
        i_n = jnp.dot(xs, win[...], preferred_element_type=jnp.float32) + bin_[...]
        h_r = jnp.dot(h, whr[...], preferred_element_type=jnp.float32) + bhr[...]
        h_z = jnp.dot(h, whz[...], preferred_element_type=jnp.float32) + bhz[...]
        h_n = jnp.dot(h, whn[...], preferred_element_type=jnp.float32) + bhn[...]
        r = jax.nn.sigmoid(i_r + h_r)
        z = jax.nn.sigmoid(i_z + h_z)
        n = jnp.tanh(i_n + r * h_n)
        return (1.0 - z) * n + z * h

    h = lax.fori_loop(0, _T, step, jnp.zeros((nb, _H), jnp.float32))
    p = jnp.dot(h, w1h[...], preferred_element_type=jnp.float32)
    p = p + jnp.dot(st_ref[...], w1s[...], preferred_element_type=jnp.float32)
    out_ref[...] = p + b1[...]


def _gru(seq_t, node_static, wiT, whT, b_ih, b_hh, w1h, w1s, b1):
    nb = 1000
    grid = _N // nb
    full = lambda r, c: pl.BlockSpec((r, c), lambda i: (0, 0))
    specs = [
        pl.BlockSpec((_T, nb, _SEQ_IN), lambda i: (0, i, 0)),
        pl.BlockSpec((nb, _STATIC), lambda i: (i, 0)),
        full(_SEQ_IN, _H), full(_SEQ_IN, _H), full(_SEQ_IN, _H),
        full(_H, _H), full(_H, _H), full(_H, _H),
        full(1, _H), full(1, _H), full(1, _H),
        full(1, _H), full(1, _H), full(1, _H),
        full(_H, _EMB), full(_STATIC, _EMB), full(1, _EMB),
    ]
    args = [seq_t, node_static,
            wiT[:, :_H], wiT[:, _H:2 * _H], wiT[:, 2 * _H:],
            whT[:, :_H], whT[:, _H:2 * _H], whT[:, 2 * _H:],
            b_ih[:_H].reshape(1, -1), b_ih[_H:2 * _H].reshape(1, -1),
            b_ih[2 * _H:].reshape(1, -1),
            b_hh[:_H].reshape(1, -1), b_hh[_H:2 * _H].reshape(1, -1),
            b_hh[2 * _H:].reshape(1, -1),
            w1h, w1s, b1]
    return pl.pallas_call(
        _gru_body,
        grid=(grid,),
        in_specs=specs,
        out_specs=pl.BlockSpec((nb, _EMB), lambda i: (i, 0)),
        out_shape=jax.ShapeDtypeStruct((_N, _EMB), jnp.float32),
    )(*args)


def _make_sc_gather(row_shape, dtype, ch, n0, n1):
    ncht = _EPAD // ch
    assert 16 * (n0 + n1) == ncht and n0 % 8 == 0 and n1 % 8 == 0
    maxn = max(n0, n1)

    @functools.partial(
        pl.kernel,
        out_type=jax.ShapeDtypeStruct((ncht, ch) + row_shape, dtype),
        mesh=_mesh,
        scratch_types=[
            pltpu.VMEM((maxn, ch), jnp.int32),
            pltpu.VMEM((ch,) + row_shape, dtype),
            pltpu.VMEM((ch,) + row_shape, dtype),
            pltpu.SemaphoreType.DMA,
            pltpu.SemaphoreType.DMA,
            pltpu.SemaphoreType.DMA,
            pltpu.SemaphoreType.DMA,
        ],
    )
    def gather(tbl, idx, out, idxbuf, rb0, rb1, gs0, gs1, ws0, ws1):
        c = lax.axis_index("c")
        s = lax.axis_index("s")

        def run(nc, base):
            pltpu.sync_copy(idx.at[pl.ds(base, nc)], idxbuf.at[pl.ds(0, nc)])
            pltpu.async_copy(tbl.at[idxbuf.at[0]], rb0, gs0)
            pltpu.async_copy(tbl.at[idxbuf.at[1]], rb1, gs1)

            def step(i, carry):
                j = 2 * i
                pltpu.make_async_copy(tbl.at[idxbuf.at[0]], rb0, gs0).wait()
                pltpu.async_copy(rb0, out.at[base + j], ws0)
                pltpu.make_async_copy(tbl.at[idxbuf.at[0]], rb1, gs1).wait()
                pltpu.async_copy(rb1, out.at[base + j + 1], ws1)

                @pl.when(j + 2 < nc)
                def _():
                    pltpu.make_async_copy(rb0, out.at[base + j], ws0).wait()
                    pltpu.async_copy(tbl.at[idxbuf.at[j + 2]], rb0, gs0)
                    pltpu.make_async_copy(rb1, out.at[base + j + 1], ws1).wait()
                    pltpu.async_copy(tbl.at[idxbuf.at[j + 3]], rb1, gs1)

                return carry

            lax.fori_loop(0, nc // 2, step, 0)
            pltpu.make_async_copy(rb0, out.at[base + nc - 2], ws0).wait()
            pltpu.make_async_copy(rb1, out.at[base + nc - 1], ws1).wait()

        @pl.when(c == 0)
        def _():
            run(n0, s * n0)

        @pl.when(c == 1)
        def _():
            run(n1, 16 * n0 + s * n1)

    return gather


_N0, _N1 = 48, 112
_gather128 = _make_sc_gather((_EMB,), jnp.float32, 128, _N0, _N1)
_gather512 = _make_sc_gather((_MLP_H // 2,), jnp.float32, 128, _N0, _N1)


@functools.partial(
    pl.kernel,
    out_type=jax.ShapeDtypeStruct((2, _NPAD, _EMB), jnp.float32),
    mesh=_mesh,
    scratch_types=[
        pltpu.VMEM((_CH, _EMB), jnp.float32),
        pltpu.VMEM((_NCH, _CH), jnp.int32),
        pltpu.VMEM_SHARED((_NPAD, _EMB), jnp.float32),
    ],
)
def _scatter_add(m3, idx3, zeros, out, mbuf, idxbuf, acc):
    c = lax.axis_index("c")
    s = lax.axis_index("s")
    w = s * 2 + c
    rpt = _NPAD // 16
    pltpu.sync_copy(zeros.at[pl.ds(s * rpt, rpt)], acc.at[pl.ds(s * rpt, rpt)])
    plsc.subcore_barrier()
    pltpu.sync_copy(idx3.at[w], idxbuf)

    def step(j, carry):
        pltpu.sync_copy(m3.at[w, j], mbuf)
        pltpu.sync_copy(mbuf, acc.at[idxbuf.at[j]], add=True)
        return carry

    lax.fori_loop(0, _NCH, step, 0)
    plsc.subcore_barrier()
    pltpu.sync_copy(acc.at[pl.ds(s * rpt, rpt)], out.at[c, pl.ds(s * rpt, rpt)])


def _mlp_body(ps_ref, ea_ref, w1b, w2, b2, out_ref):
    t = ps_ref[...] + jnp.dot(ea_ref[...], w1b[...],
                              preferred_element_type=jnp.float32)
    t = jax.nn.relu(t)
    out_ref[...] = jnp.dot(t, w2[...], preferred_element_type=jnp.float32) + b2[...]


def _edge_mlp(ps, ea, w1b, w2, b2):
    blk = 2048
    grid = _EPAD // blk
    full = lambda r, c: pl.BlockSpec((r, c), lambda i: (0, 0))
    return pl.pallas_call(
        _mlp_body,
        grid=(grid,),
        in_specs=[
            pl.BlockSpec((blk, _EMB), lambda i: (i, 0)),
            pl.BlockSpec((blk, 8), lambda i: (i, 0)),
            full(8, _EMB), full(_EMB, _EMB), full(1, _EMB),
        ],
        out_specs=pl.BlockSpec((blk, _EMB), lambda i: (i, 0)),
        out_shape=jax.ShapeDtypeStruct((_EPAD, _EMB), jnp.float32),
    )(ps, ea, w1b, w2, b2)


def _rne_bf16_bits(x):
    u = lax.bitcast_convert_type(x, jnp.uint32)
    return (u + jnp.uint32(0x7FFF) + ((u >> 16) & jnp.uint32(1))) >> 16


def _pack_bf16(x):
    half = x.shape[1] // 2
    lo = _rne_bf16_bits(x[:, :half])
    hi = _rne_bf16_bits(x[:, half:])
    return lax.bitcast_convert_type(lo | (hi << 16), jnp.float32)


def _unpack_bf16(x):
    u = lax.bitcast_convert_type(x, jnp.uint32)
    lo = lax.bitcast_convert_type(u << 16, jnp.float32)
    hi = lax.bitcast_convert_type(u & jnp.uint32(0xFFFF0000), jnp.float32)
    return lo, hi


def _ab_body(u2_ref, wla, wlb, bl1, a_ref, b_ref):
    u = u2_ref[0] + u2_ref[1]
    a = jnp.dot(u, wla[...], preferred_element_type=jnp.float32) + bl1[...]
    b = jnp.dot(u, wlb[...], preferred_element_type=jnp.float32)
    a_ref[...] = _pack_bf16(a)
    b_ref[...] = _pack_bf16(b)


def _ab(u2, wla, wlb, bl1):
    nb = 1000
    grid = _N // nb
    full = lambda r, c: pl.BlockSpec((r, c), lambda i: (0, 0))
    return pl.pallas_call(
        _ab_body,
        grid=(grid,),
        in_specs=[
            pl.BlockSpec((2, nb, _EMB), lambda i: (0, i, 0)),
            full(_EMB, _MLP_H), full(_EMB, _MLP_H), full(1, _MLP_H),
        ],
        out_specs=[
            pl.BlockSpec((nb, _MLP_H // 2), lambda i: (i, 0)),
            pl.BlockSpec((nb, _MLP_H // 2), lambda i: (i, 0)),
        ],
        out_shape=[
            jax.ShapeDtypeStruct((_N, _MLP_H // 2), jnp.float32),
            jax.ShapeDtypeStruct((_N, _MLP_H // 2), jnp.float32),
        ],
    )(u2, wla, wlb, bl1)


def _pred_body(s1_ref, s2_ref, ea_ref, wlc_lo, wlc_hi, wl2_lo, wl2_hi,
               bl2, out_ref):
    s1_lo, s1_hi = _unpack_bf16(s1_ref[...])
    s2_lo, s2_hi = _unpack_bf16(s2_ref[...])
    ea = ea_ref[...]
    h_lo = s1_lo + s2_lo + jnp.dot(ea, wlc_lo[...],
                                   preferred_element_type=jnp.float32)
    h_hi = s1_hi + s2_hi + jnp.dot(ea, wlc_hi[...],
                                   preferred_element_type=jnp.float32)
    h_lo = jax.nn.relu(h_lo)
    h_hi = jax.nn.relu(h_hi)
    logit = (jnp.dot(h_lo, wl2_lo[...], preferred_element_type=jnp.float32)
             + jnp.dot(h_hi, wl2_hi[...], preferred_element_type=jnp.float32)
             + bl2[...])
    out_ref[...] = jax.nn.sigmoid(logit)


def _pred(s1, s2, ea, wlc, wl2, bl2):
    blk = 1024
    grid = _EPAD // blk
    hh = _MLP_H // 2
    full = lambda r, c: pl.BlockSpec((r, c), lambda i: (0, 0))
    return pl.pallas_call(
        _pred_body,
        grid=(grid,),
        in_specs=[
            pl.BlockSpec((blk, hh), lambda i: (i, 0)),
            pl.BlockSpec((blk, hh), lambda i: (i, 0)),
            pl.BlockSpec((blk, 8), lambda i: (i, 0)),
            full(8, hh), full(8, hh), full(hh, 1), full(hh, 1), full(1, 1),
        ],
        out_specs=pl.BlockSpec((blk, 1), lambda i: (i, 0)),
        out_shape=jax.ShapeDtypeStruct((_EPAD, 1), jnp.float32),
    )(s1, s2, ea, wlc[:, :hh], wlc[:, hh:], wl2[:hh], wl2[hh:], bl2)


def kernel(player_seq, node_static, edge_index, edge_attr,
           W_ih, W_hh, b_ih, b_hh, W1, b1, W2, b2, Wl1, bl1, Wl2, bl2):
    f32 = jnp.float32
    pad = _EPAD - _E

    seq_t = jnp.transpose(player_seq, (1, 0, 2))
    src_p = jnp.concatenate([edge_index[0], jnp.zeros((pad,), jnp.int32)])
    dst_p = jnp.concatenate([edge_index[1], jnp.full((pad,), _N, jnp.int32)])
    src3 = src_p.reshape(_NT, _NCH, _CH)
    dst3 = dst_p.reshape(_NT, _NCH, _CH)
    src2 = src_p.reshape(_EPAD // _CH, _CH)
    dst2 = dst_p.reshape(_EPAD // _CH, _CH)
    ea_p = jnp.pad(edge_attr, ((0, pad), (0, 8 - _EDGE_F)))

    node_in = _H + _STATIC
    w1h = W1[:_H]
    w1s = W1[_H:node_in]
    w1b = jnp.concatenate([W1[node_in:], jnp.zeros((8 - _EDGE_F, _EMB), f32)])
    wla = Wl1[:_EMB]
    wlb = Wl1[_EMB:2 * _EMB]
    wlc = jnp.concatenate([Wl1[2 * _EMB:], jnp.zeros((8 - _EDGE_F, _MLP_H), f32)])

    p = _gru(seq_t, node_static, W_ih.T, W_hh.T, b_ih, b_hh,
             w1h, w1s, b1.reshape(1, -1))

    ps = _gather128(p, src2).reshape(_EPAD, _EMB)

    m = _edge_mlp(ps, ea_p, w1b, W2, b2.reshape(1, -1))

    m3 = m.reshape(_NT, _NCH, _CH, _EMB)
    u2 = _scatter_add(m3, dst3, jnp.zeros((_NPAD, _EMB), f32))

    a, b = _ab(u2, wla, wlb, bl1.reshape(1, -1))

    s1 = _gather512(a, src2).reshape(_EPAD, _MLP_H // 2)
    s2 = _gather512(b, dst2).reshape(_EPAD, _MLP_H // 2)

    pred = _pred(s1, s2, ea_p, wlc, Wl2, bl2.reshape(1, 1))
    return pred[:_E]


# speedup vs baseline: 1.0232x; 1.0232x over previous
import functools

import jax
import jax.numpy as jnp
from jax import lax
from jax.experimental import pallas as pl
from jax.experimental.pallas import tpu as pltpu
from jax.experimental.pallas import tpu_sc as plsc

_N = 10000
_E = 320000
_T = 20
_SEQ_IN = 5
_H = 32
_STATIC = 7
_EMB = 128
_EDGE_F = 5
_MLP_H = 512

_NT = 32
_CH = 128
_NCH = 80
_EPAD = _NT * _NCH * _CH
_NPAD = 10112

_mesh = plsc.VectorSubcoreMesh(
    core_axis_name="c", subcore_axis_name="s", num_cores=2, num_subcores=16)


def _gru_body(seq_ref, st_ref, wir, wiz, win, whr, whz, whn,
              bir, biz, bin_, bhr, bhz, bhn, w1h, w1s, b1, out_ref):
    nb = st_ref.shape[0]

    def step(t, h):
        xs = seq_ref[t]
        i_r = jnp.dot(xs, wir[...], preferred_element_type=jnp.float32) + bir[...]
        i_z = jnp.dot(xs, wiz[...], preferred_element_type=jnp.float32) + biz[...]
        i_n = jnp.dot(xs, win[...], preferred_element_type=jnp.float32) + bin_[...]
        h_r = jnp.dot(h, whr[...], preferred_element_type=jnp.float32) + bhr[...]
        h_z = jnp.dot(h, whz[...], preferred_element_type=jnp.float32) + bhz[...]
        h_n = jnp.dot(h, whn[...], preferred_element_type=jnp.float32) + bhn[...]
        r = jax.nn.sigmoid(i_r + h_r)
        z = jax.nn.sigmoid(i_z + h_z)
        n = jnp.tanh(i_n + r * h_n)
        return (1.0 - z) * n + z * h

    h = lax.fori_loop(0, _T, step, jnp.zeros((nb, _H), jnp.float32))
    p = jnp.dot(h, w1h[...], preferred_element_type=jnp.float32)
    p = p + jnp.dot(st_ref[...], w1s[...], preferred_element_type=jnp.float32)
    out_ref[...] = p + b1[...]


def _gru(seq_t, node_static, wiT, whT, b_ih, b_hh, w1h, w1s, b1):
    nb = 1000
    grid = _N // nb
    full = lambda r, c: pl.BlockSpec((r, c), lambda i: (0, 0))
    specs = [
        pl.BlockSpec((_T, nb, _SEQ_IN), lambda i: (0, i, 0)),
        pl.BlockSpec((nb, _STATIC), lambda i: (i, 0)),
        full(_SEQ_IN, _H), full(_SEQ_IN, _H), full(_SEQ_IN, _H),
        full(_H, _H), full(_H, _H), full(_H, _H),
        full(1, _H), full(1, _H), full(1, _H),
        full(1, _H), full(1, _H), full(1, _H),
        full(_H, _EMB), full(_STATIC, _EMB), full(1, _EMB),
    ]
    args = [seq_t, node_static,
            wiT[:, :_H], wiT[:, _H:2 * _H], wiT[:, 2 * _H:],
            whT[:, :_H], whT[:, _H:2 * _H], whT[:, 2 * _H:],
            b_ih[:_H].reshape(1, -1), b_ih[_H:2 * _H].reshape(1, -1),
            b_ih[2 * _H:].reshape(1, -1),
            b_hh[:_H].reshape(1, -1), b_hh[_H:2 * _H].reshape(1, -1),
            b_hh[2 * _H:].reshape(1, -1),
            w1h, w1s, b1]
    return pl.pallas_call(
        _gru_body,
        grid=(grid,),
        in_specs=specs,
        out_specs=pl.BlockSpec((nb, _EMB), lambda i: (i, 0)),
        out_shape=jax.ShapeDtypeStruct((_N, _EMB), jnp.float32),
    )(*args)


def _make_sc_gather(row_shape, dtype, ch, n0, n1):
    ncht = _EPAD // ch
    assert 16 * (n0 + n1) == ncht and n0 % 8 == 0 and n1 % 8 == 0
    maxn = max(n0, n1)

    @functools.partial(
        pl.kernel,
        out_type=jax.ShapeDtypeStruct((ncht, ch) + row_shape, dtype),
        mesh=_mesh,
        scratch_types=[
            pltpu.VMEM((maxn, ch), jnp.int32),
            pltpu.VMEM((ch,) + row_shape, dtype),
            pltpu.VMEM((ch,) + row_shape, dtype),
            pltpu.SemaphoreType.DMA,
            pltpu.SemaphoreType.DMA,
            pltpu.SemaphoreType.DMA,
            pltpu.SemaphoreType.DMA,
        ],
    )
    def gather(tbl, idx, out, idxbuf, rb0, rb1, gs0, gs1, ws0, ws1):
        c = lax.axis_index("c")
        s = lax.axis_index("s")

        def run(nc, base):
            pltpu.sync_copy(idx.at[pl.ds(base, nc)], idxbuf.at[pl.ds(0, nc)])
            pltpu.async_copy(tbl.at[idxbuf.at[0]], rb0, gs0)
            pltpu.async_copy(tbl.at[idxbuf.at[1]], rb1, gs1)

            def step(i, carry):
                j = 2 * i
                pltpu.make_async_copy(tbl.at[idxbuf.at[0]], rb0, gs0).wait()
                pltpu.async_copy(rb0, out.at[base + j], ws0)
                pltpu.make_async_copy(tbl.at[idxbuf.at[0]], rb1, gs1).wait()
                pltpu.async_copy(rb1, out.at[base + j + 1], ws1)

                @pl.when(j + 2 < nc)
                def _():
                    pltpu.make_async_copy(rb0, out.at[base + j], ws0).wait()
                    pltpu.async_copy(tbl.at[idxbuf.at[j + 2]], rb0, gs0)
                    pltpu.make_async_copy(rb1, out.at[base + j + 1], ws1).wait()
                    pltpu.async_copy(tbl.at[idxbuf.at[j + 3]], rb1, gs1)

                return carry

            lax.fori_loop(0, nc // 2, step, 0)
            pltpu.make_async_copy(rb0, out.at[base + nc - 2], ws0).wait()
            pltpu.make_async_copy(rb1, out.at[base + nc - 1], ws1).wait()

        @pl.when(c == 0)
        def _():
            run(n0, s * n0)

        @pl.when(c == 1)
        def _():
            run(n1, 16 * n0 + s * n1)

    return gather


_N0, _N1 = 112, 48
_gather128 = _make_sc_gather((_EMB,), jnp.float32, 128, _N0, _N1)
_gather512 = _make_sc_gather((_MLP_H // 2,), jnp.float32, 128, _N0, _N1)


@functools.partial(
    pl.kernel,
    out_type=jax.ShapeDtypeStruct((2, _NPAD, _EMB), jnp.float32),
    mesh=_mesh,
    scratch_types=[
        pltpu.VMEM((_CH, _EMB), jnp.float32),
        pltpu.VMEM((_NCH, _CH), jnp.int32),
        pltpu.VMEM_SHARED((_NPAD, _EMB), jnp.float32),
    ],
)
def _scatter_add(m3, idx3, zeros, out, mbuf, idxbuf, acc):
    c = lax.axis_index("c")
    s = lax.axis_index("s")
    w = s * 2 + c
    rpt = _NPAD // 16
    pltpu.sync_copy(zeros.at[pl.ds(s * rpt, rpt)], acc.at[pl.ds(s * rpt, rpt)])
    plsc.subcore_barrier()
    pltpu.sync_copy(idx3.at[w], idxbuf)

    def step(j, carry):
        pltpu.sync_copy(m3.at[w, j], mbuf)
        pltpu.sync_copy(mbuf, acc.at[idxbuf.at[j]], add=True)
        return carry

    lax.fori_loop(0, _NCH, step, 0)
    plsc.subcore_barrier()
    pltpu.sync_copy(acc.at[pl.ds(s * rpt, rpt)], out.at[c, pl.ds(s * rpt, rpt)])


def _mlp_body(ps_ref, ea_ref, w1b, w2, b2, out_ref):
    t = ps_ref[...] + jnp.dot(ea_ref[...], w1b[...],
                              preferred_element_type=jnp.float32)
    t = jax.nn.relu(t)
    out_ref[...] = jnp.dot(t, w2[...], preferred_element_type=jnp.float32) + b2[...]


def _edge_mlp(ps, ea, w1b, w2, b2):
    blk = 2048
    grid = _EPAD // blk
    full = lambda r, c: pl.BlockSpec((r, c), lambda i: (0, 0))
    return pl.pallas_call(
        _mlp_body,
        grid=(grid,),
        in_specs=[
            pl.BlockSpec((blk, _EMB), lambda i: (i, 0)),
            pl.BlockSpec((blk, 8), lambda i: (i, 0)),
            full(8, _EMB), full(_EMB, _EMB), full(1, _EMB),
        ],
        out_specs=pl.BlockSpec((blk, _EMB), lambda i: (i, 0)),
        out_shape=jax.ShapeDtypeStruct((_EPAD, _EMB), jnp.float32),
    )(ps, ea, w1b, w2, b2)


def _rne_bf16_bits(x):
    u = lax.bitcast_convert_type(x, jnp.uint32)
    return (u + jnp.uint32(0x7FFF) + ((u >> 16) & jnp.uint32(1))) >> 16


def _pack_bf16(x):
    half = x.shape[1] // 2
    lo = _rne_bf16_bits(x[:, :half])
    hi = _rne_bf16_bits(x[:, half:])
    return lax.bitcast_convert_type(lo | (hi << 16), jnp.float32)


def _unpack_bf16(x):
    u = lax.bitcast_convert_type(x, jnp.uint32)
    lo = lax.bitcast_convert_type(u << 16, jnp.float32)
    hi = lax.bitcast_convert_type(u & jnp.uint32(0xFFFF0000), jnp.float32)
    return lo, hi


def _ab_body(u2_ref, wla, wlb, bl1, a_ref, b_ref):
    u = u2_ref[0] + u2_ref[1]
    a = jnp.dot(u, wla[...], preferred_element_type=jnp.float32) + bl1[...]
    b = jnp.dot(u, wlb[...], preferred_element_type=jnp.float32)
    a_ref[...] = _pack_bf16(a)
    b_ref[...] = _pack_bf16(b)


def _ab(u2, wla, wlb, bl1):
    nb = 1000
    grid = _N // nb
    full = lambda r, c: pl.BlockSpec((r, c), lambda i: (0, 0))
    return pl.pallas_call(
        _ab_body,
        grid=(grid,),
        in_specs=[
            pl.BlockSpec((2, nb, _EMB), lambda i: (0, i, 0)),
            full(_EMB, _MLP_H), full(_EMB, _MLP_H), full(1, _MLP_H),
        ],
        out_specs=[
            pl.BlockSpec((nb, _MLP_H // 2), lambda i: (i, 0)),
            pl.BlockSpec((nb, _MLP_H // 2), lambda i: (i, 0)),
        ],
        out_shape=[
            jax.ShapeDtypeStruct((_N, _MLP_H // 2), jnp.float32),
            jax.ShapeDtypeStruct((_N, _MLP_H // 2), jnp.float32),
        ],
    )(u2, wla, wlb, bl1)


def _pred_body(s1_ref, s2_ref, ea_ref, wlc_lo, wlc_hi, wl2_lo, wl2_hi,
               bl2, out_ref):
    s1_lo, s1_hi = _unpack_bf16(s1_ref[...])
    s2_lo, s2_hi = _unpack_bf16(s2_ref[...])
    ea = ea_ref[...]
    h_lo = s1_lo + s2_lo + jnp.dot(ea, wlc_lo[...],
                                   preferred_element_type=jnp.float32)
    h_hi = s1_hi + s2_hi + jnp.dot(ea, wlc_hi[...],
                                   preferred_element_type=jnp.float32)
    h_lo = jax.nn.relu(h_lo)
    h_hi = jax.nn.relu(h_hi)
    logit = (jnp.dot(h_lo, wl2_lo[...], preferred_element_type=jnp.float32)
             + jnp.dot(h_hi, wl2_hi[...], preferred_element_type=jnp.float32)
             + bl2[...])
    out_ref[...] = jax.nn.sigmoid(logit)


def _pred(s1, s2, ea, wlc, wl2, bl2):
    blk = 1024
    grid = _EPAD // blk
    hh = _MLP_H // 2
    full = lambda r, c: pl.BlockSpec((r, c), lambda i: (0, 0))
    return pl.pallas_call(
        _pred_body,
        grid=(grid,),
        in_specs=[
            pl.BlockSpec((blk, hh), lambda i: (i, 0)),
            pl.BlockSpec((blk, hh), lambda i: (i, 0)),
            pl.BlockSpec((blk, 8), lambda i: (i, 0)),
            full(8, hh), full(8, hh), full(hh, 1), full(hh, 1), full(1, 1),
        ],
        out_specs=pl.BlockSpec((blk, 1), lambda i: (i, 0)),
        out_shape=jax.ShapeDtypeStruct((_EPAD, 1), jnp.float32),
    )(s1, s2, ea, wlc[:, :hh], wlc[:, hh:], wl2[:hh], wl2[hh:], bl2)


def kernel(player_seq, node_static, edge_index, edge_attr,
           W_ih, W_hh, b_ih, b_hh, W1, b1, W2, b2, Wl1, bl1, Wl2, bl2):
    f32 = jnp.float32
    pad = _EPAD - _E

    seq_t = jnp.transpose(player_seq, (1, 0, 2))
    src_p = jnp.concatenate([edge_index[0], jnp.zeros((pad,), jnp.int32)])
    dst_p = jnp.concatenate([edge_index[1], jnp.full((pad,), _N, jnp.int32)])
    src3 = src_p.reshape(_NT, _NCH, _CH)
    dst3 = dst_p.reshape(_NT, _NCH, _CH)
    src2 = src_p.reshape(_EPAD // _CH, _CH)
    dst2 = dst_p.reshape(_EPAD // _CH, _CH)
    ea_p = jnp.pad(edge_attr, ((0, pad), (0, 8 - _EDGE_F)))

    node_in = _H + _STATIC
    w1h = W1[:_H]
    w1s = W1[_H:node_in]
    w1b = jnp.concatenate([W1[node_in:], jnp.zeros((8 - _EDGE_F, _EMB), f32)])
    wla = Wl1[:_EMB]
    wlb = Wl1[_EMB:2 * _EMB]
    wlc = jnp.concatenate([Wl1[2 * _EMB:], jnp.zeros((8 - _EDGE_F, _MLP_H), f32)])

    p = _gru(seq_t, node_static, W_ih.T, W_hh.T, b_ih, b_hh,
             w1h, w1s, b1.reshape(1, -1))

    ps = _gather128(p, src2).reshape(_EPAD, _EMB)

    m = _edge_mlp(ps, ea_p, w1b, W2, b2.reshape(1, -1))

    m3 = m.reshape(_NT, _NCH, _CH, _EMB)
    u2 = _scatter_add(m3, dst3, jnp.zeros((_NPAD, _EMB), f32))

    a, b = _ab(u2, wla, wlb, bl1.reshape(1, -1))

    s1 = _gather512(a, src2).reshape(_EPAD, _MLP_H // 2)
    s2 = _gather512(b, dst2).reshape(_EPAD, _MLP_H // 2)

    pred = _pred(s1, s2, ea_p, wlc, Wl2, bl2.reshape(1, 1))
    return pred[:_E]


# speedup vs baseline: 1.2491x; 1.2208x over previous
import functools

import jax
import jax.numpy as jnp
from jax import lax
from jax.experimental import pallas as pl
from jax.experimental.pallas import tpu as pltpu
from jax.experimental.pallas import tpu_sc as plsc

_N = 10000
_E = 320000
_T = 20
_SEQ_IN = 5
_H = 32
_STATIC = 7
_EMB = 128
_EDGE_F = 5
_MLP_H = 512

_NT = 32
_CH = 128
_NCH = 80
_EPAD = _NT * _NCH * _CH
_NPAD = 10112

_mesh = plsc.VectorSubcoreMesh(
    core_axis_name="c", subcore_axis_name="s", num_cores=2, num_subcores=16)


def _gru_body(seq_ref, st_ref, wir, wiz, win, whr, whz, whn,
              bir, biz, bin_, bhr, bhz, bhn, w1h, w1s, b1, out_ref):
    nb = st_ref.shape[0]

    def step(t, h):
        xs = seq_ref[t]
        i_r = jnp.dot(xs, wir[...], preferred_element_type=jnp.float32) + bir[...]
        i_z = jnp.dot(xs, wiz[...], preferred_element_type=jnp.float32) + biz[...]
        i_n = jnp.dot(xs, win[...], preferred_element_type=jnp.float32) + bin_[...]
        h_r = jnp.dot(h, whr[...], preferred_element_type=jnp.float32) + bhr[...]
        h_z = jnp.dot(h, whz[...], preferred_element_type=jnp.float32) + bhz[...]
        h_n = jnp.dot(h, whn[...], preferred_element_type=jnp.float32) + bhn[...]
        r = jax.nn.sigmoid(i_r + h_r)
        z = jax.nn.sigmoid(i_z + h_z)
        n = jnp.tanh(i_n + r * h_n)
        return (1.0 - z) * n + z * h

    h = lax.fori_loop(0, _T, step, jnp.zeros((nb, _H), jnp.float32))
    p = jnp.dot(h, w1h[...], preferred_element_type=jnp.float32)
    p = p + jnp.dot(st_ref[...], w1s[...], preferred_element_type=jnp.float32)
    out_ref[...] = p + b1[...]


def _gru(seq_t, node_static, wiT, whT, b_ih, b_hh, w1h, w1s, b1):
    nb = 1000
    grid = _N // nb
    full = lambda r, c: pl.BlockSpec((r, c), lambda i: (0, 0))
    specs = [
        pl.BlockSpec((_T, nb, _SEQ_IN), lambda i: (0, i, 0)),
        pl.BlockSpec((nb, _STATIC), lambda i: (i, 0)),
        full(_SEQ_IN, _H), full(_SEQ_IN, _H), full(_SEQ_IN, _H),
        full(_H, _H), full(_H, _H), full(_H, _H),
        full(1, _H), full(1, _H), full(1, _H),
        full(1, _H), full(1, _H), full(1, _H),
        full(_H, _EMB), full(_STATIC, _EMB), full(1, _EMB),
    ]
    args = [seq_t, node_static,
            wiT[:, :_H], wiT[:, _H:2 * _H], wiT[:, 2 * _H:],
            whT[:, :_H], whT[:, _H:2 * _H], whT[:, 2 * _H:],
            b_ih[:_H].reshape(1, -1), b_ih[_H:2 * _H].reshape(1, -1),
            b_ih[2 * _H:].reshape(1, -1),
            b_hh[:_H].reshape(1, -1), b_hh[_H:2 * _H].reshape(1, -1),
            b_hh[2 * _H:].reshape(1, -1),
            w1h, w1s, b1]
    return pl.pallas_call(
        _gru_body,
        grid=(grid,),
        in_specs=specs,
        out_specs=pl.BlockSpec((nb, _EMB), lambda i: (i, 0)),
        out_shape=jax.ShapeDtypeStruct((_N, _EMB), jnp.float32),
    )(*args)


def _make_sc_gather(row_shape, dtype, ch, n0, n1):
    ncht = _EPAD // ch
    assert 16 * (n0 + n1) == ncht and n0 % 8 == 0 and n1 % 8 == 0
    maxn = max(n0, n1)

    @functools.partial(
        pl.kernel,
        out_type=jax.ShapeDtypeStruct((ncht, ch) + row_shape, dtype),
        mesh=_mesh,
        scratch_types=[
            pltpu.VMEM((maxn, ch), jnp.int32),
            pltpu.VMEM((ch,) + row_shape, dtype),
            pltpu.VMEM((ch,) + row_shape, dtype),
            pltpu.SemaphoreType.DMA,
            pltpu.SemaphoreType.DMA,
            pltpu.SemaphoreType.DMA,
            pltpu.SemaphoreType.DMA,
        ],
    )
    def gather(tbl, idx, out, idxbuf, rb0, rb1, gs0, gs1, ws0, ws1):
        c = lax.axis_index("c")
        s = lax.axis_index("s")

        def run(nc, base):
            pltpu.sync_copy(idx.at[pl.ds(base, nc)], idxbuf.at[pl.ds(0, nc)])
            pltpu.async_copy(tbl.at[idxbuf.at[0]], rb0, gs0)
            pltpu.async_copy(tbl.at[idxbuf.at[1]], rb1, gs1)

            def step(i, carry):
                j = 2 * i
                pltpu.make_async_copy(tbl.at[idxbuf.at[0]], rb0, gs0).wait()
                pltpu.async_copy(rb0, out.at[base + j], ws0)
                pltpu.make_async_copy(tbl.at[idxbuf.at[0]], rb1, gs1).wait()
                pltpu.async_copy(rb1, out.at[base + j + 1], ws1)

                @pl.when(j + 2 < nc)
                def _():
                    pltpu.make_async_copy(rb0, out.at[base + j], ws0).wait()
                    pltpu.async_copy(tbl.at[idxbuf.at[j + 2]], rb0, gs0)
                    pltpu.make_async_copy(rb1, out.at[base + j + 1], ws1).wait()
                    pltpu.async_copy(tbl.at[idxbuf.at[j + 3]], rb1, gs1)

                return carry

            lax.fori_loop(0, nc // 2, step, 0)
            pltpu.make_async_copy(rb0, out.at[base + nc - 2], ws0).wait()
            pltpu.make_async_copy(rb1, out.at[base + nc - 1], ws1).wait()

        @pl.when(c == 0)
        def _():
            run(n0, s * n0)

        @pl.when(c == 1)
        def _():
            run(n1, 16 * n0 + s * n1)

    return gather


_N0, _N1 = 80, 80
_gather512 = _make_sc_gather((_MLP_H // 2,), jnp.float32, 128, _N0, _N1)
_PPAD = 10112


@functools.partial(
    pl.kernel,
    out_type=jax.ShapeDtypeStruct((_EPAD // _CH, _CH, _EMB), jnp.float32),
    mesh=_mesh,
    scratch_types=[
        pltpu.VMEM((_NCH, _CH), jnp.int32),
        pltpu.VMEM((_CH, _EMB), jnp.float32),
        pltpu.VMEM((_CH, _EMB), jnp.float32),
        pltpu.VMEM_SHARED((_PPAD, _EMB), jnp.float32),
        pltpu.SemaphoreType.DMA,
        pltpu.SemaphoreType.DMA,
        pltpu.SemaphoreType.DMA,
        pltpu.SemaphoreType.DMA,
    ],
)
def _gather128(tbl, idx, out, idxbuf, rb0, rb1, ptbl, gs0, gs1, ws0, ws1):
    c = lax.axis_index("c")
    s = lax.axis_index("s")
    rpt = _PPAD // 16
    pltpu.sync_copy(tbl.at[pl.ds(s * rpt, rpt)], ptbl.at[pl.ds(s * rpt, rpt)])
    plsc.subcore_barrier()
    w = s * 2 + c
    base = w * _NCH
    pltpu.sync_copy(idx.at[pl.ds(base, _NCH)], idxbuf)
    pltpu.async_copy(ptbl.at[idxbuf.at[0]], rb0, gs0)
    pltpu.async_copy(ptbl.at[idxbuf.at[1]], rb1, gs1)

    def step(i, carry):
        j = 2 * i
        pltpu.make_async_copy(ptbl.at[idxbuf.at[0]], rb0, gs0).wait()
        pltpu.async_copy(rb0, out.at[base + j], ws0)
        pltpu.make_async_copy(ptbl.at[idxbuf.at[0]], rb1, gs1).wait()
        pltpu.async_copy(rb1, out.at[base + j + 1], ws1)

        @pl.when(j + 2 < _NCH)
        def _():
            pltpu.make_async_copy(rb0, out.at[base + j], ws0).wait()
            pltpu.async_copy(ptbl.at[idxbuf.at[j + 2]], rb0, gs0)
            pltpu.make_async_copy(rb1, out.at[base + j + 1], ws1).wait()
            pltpu.async_copy(ptbl.at[idxbuf.at[j + 3]], rb1, gs1)

        return carry

    lax.fori_loop(0, _NCH // 2, step, 0)
    pltpu.make_async_copy(rb0, out.at[base + _NCH - 2], ws0).wait()
    pltpu.make_async_copy(rb1, out.at[base + _NCH - 1], ws1).wait()


@functools.partial(
    pl.kernel,
    out_type=jax.ShapeDtypeStruct((2, _NPAD, _EMB), jnp.float32),
    mesh=_mesh,
    scratch_types=[
        pltpu.VMEM((_CH, _EMB), jnp.float32),
        pltpu.VMEM((_NCH, _CH), jnp.int32),
        pltpu.VMEM_SHARED((_NPAD, _EMB), jnp.float32),
    ],
)
def _scatter_add(m3, idx3, zeros, out, mbuf, idxbuf, acc):
    c = lax.axis_index("c")
    s = lax.axis_index("s")
    w = s * 2 + c
    rpt = _NPAD // 16
    pltpu.sync_copy(zeros.at[pl.ds(s * rpt, rpt)], acc.at[pl.ds(s * rpt, rpt)])
    plsc.subcore_barrier()
    pltpu.sync_copy(idx3.at[w], idxbuf)

    def step(j, carry):
        pltpu.sync_copy(m3.at[w, j], mbuf)
        pltpu.sync_copy(mbuf, acc.at[idxbuf.at[j]], add=True)
        return carry

    lax.fori_loop(0, _NCH, step, 0)
    plsc.subcore_barrier()
    pltpu.sync_copy(acc.at[pl.ds(s * rpt, rpt)], out.at[c, pl.ds(s * rpt, rpt)])


def _mlp_body(ps_ref, ea_ref, w1b, w2, b2, out_ref):
    t = ps_ref[...] + jnp.dot(ea_ref[...], w1b[...],
                              preferred_element_type=jnp.float32)
    t = jax.nn.relu(t)
    out_ref[...] = jnp.dot(t, w2[...], preferred_element_type=jnp.float32) + b2[...]


def _edge_mlp(ps, ea, w1b, w2, b2):
    blk = 2048
    grid = _EPAD // blk
    full = lambda r, c: pl.BlockSpec((r, c), lambda i: (0, 0))
    return pl.pallas_call(
        _mlp_body,
        grid=(grid,),
        in_specs=[
            pl.BlockSpec((blk, _EMB), lambda i: (i, 0)),
            pl.BlockSpec((blk, 8), lambda i: (i, 0)),
            full(8, _EMB), full(_EMB, _EMB), full(1, _EMB),
        ],
        out_specs=pl.BlockSpec((blk, _EMB), lambda i: (i, 0)),
        out_shape=jax.ShapeDtypeStruct((_EPAD, _EMB), jnp.float32),
    )(ps, ea, w1b, w2, b2)


def _rne_bf16_bits(x):
    u = lax.bitcast_convert_type(x, jnp.uint32)
    return (u + jnp.uint32(0x7FFF) + ((u >> 16) & jnp.uint32(1))) >> 16


def _pack_bf16(x):
    half = x.shape[1] // 2
    lo = _rne_bf16_bits(x[:, :half])
    hi = _rne_bf16_bits(x[:, half:])
    return lax.bitcast_convert_type(lo | (hi << 16), jnp.float32)


def _unpack_bf16(x):
    u = lax.bitcast_convert_type(x, jnp.uint32)
    lo = lax.bitcast_convert_type(u << 16, jnp.float32)
    hi = lax.bitcast_convert_type(u & jnp.uint32(0xFFFF0000), jnp.float32)
    return lo, hi


def _ab_body(u2_ref, wla, wlb, bl1, a_ref, b_ref):
    u = u2_ref[0] + u2_ref[1]
    a = jnp.dot(u, wla[...], preferred_element_type=jnp.float32) + bl1[...]
    b = jnp.dot(u, wlb[...], preferred_element_type=jnp.float32)
    a_ref[...] = _pack_bf16(a)
    b_ref[...] = _pack_bf16(b)


def _ab(u2, wla, wlb, bl1):
    nb = 1000
    grid = _N // nb
    full = lambda r, c: pl.BlockSpec((r, c), lambda i: (0, 0))
    return pl.pallas_call(
        _ab_body,
        grid=(grid,),
        in_specs=[
            pl.BlockSpec((2, nb, _EMB), lambda i: (0, i, 0)),
            full(_EMB, _MLP_H), full(_EMB, _MLP_H), full(1, _MLP_H),
        ],
        out_specs=[
            pl.BlockSpec((nb, _MLP_H // 2), lambda i: (i, 0)),
            pl.BlockSpec((nb, _MLP_H // 2), lambda i: (i, 0)),
        ],
        out_shape=[
            jax.ShapeDtypeStruct((_N, _MLP_H // 2), jnp.float32),
            jax.ShapeDtypeStruct((_N, _MLP_H // 2), jnp.float32),
        ],
    )(u2, wla, wlb, bl1)


def _pred_body(s1_ref, s2_ref, ea_ref, wlc_lo, wlc_hi, wl2_lo, wl2_hi,
               bl2, out_ref):
    s1_lo, s1_hi = _unpack_bf16(s1_ref[...])
    s2_lo, s2_hi = _unpack_bf16(s2_ref[...])
    ea = ea_ref[...]
    h_lo = s1_lo + s2_lo + jnp.dot(ea, wlc_lo[...],
                                   preferred_element_type=jnp.float32)
    h_hi = s1_hi + s2_hi + jnp.dot(ea, wlc_hi[...],
                                   preferred_element_type=jnp.float32)
    h_lo = jax.nn.relu(h_lo)
    h_hi = jax.nn.relu(h_hi)
    logit = (jnp.dot(h_lo, wl2_lo[...], preferred_element_type=jnp.float32)
             + jnp.dot(h_hi, wl2_hi[...], preferred_element_type=jnp.float32)
             + bl2[...])
    out_ref[...] = jax.nn.sigmoid(logit)


def _pred(s1, s2, ea, wlc, wl2, bl2):
    blk = 1024
    grid = _EPAD // blk
    hh = _MLP_H // 2
    full = lambda r, c: pl.BlockSpec((r, c), lambda i: (0, 0))
    return pl.pallas_call(
        _pred_body,
        grid=(grid,),
        in_specs=[
            pl.BlockSpec((blk, hh), lambda i: (i, 0)),
            pl.BlockSpec((blk, hh), lambda i: (i, 0)),
            pl.BlockSpec((blk, 8), lambda i: (i, 0)),
            full(8, hh), full(8, hh), full(hh, 1), full(hh, 1), full(1, 1),
        ],
        out_specs=pl.BlockSpec((blk, 1), lambda i: (i, 0)),
        out_shape=jax.ShapeDtypeStruct((_EPAD, 1), jnp.float32),
    )(s1, s2, ea, wlc[:, :hh], wlc[:, hh:], wl2[:hh], wl2[hh:], bl2)


def kernel(player_seq, node_static, edge_index, edge_attr,
           W_ih, W_hh, b_ih, b_hh, W1, b1, W2, b2, Wl1, bl1, Wl2, bl2):
    f32 = jnp.float32
    pad = _EPAD - _E

    seq_t = jnp.transpose(player_seq, (1, 0, 2))
    src_p = jnp.concatenate([edge_index[0], jnp.zeros((pad,), jnp.int32)])
    dst_p = jnp.concatenate([edge_index[1], jnp.full((pad,), _N, jnp.int32)])
    src3 = src_p.reshape(_NT, _NCH, _CH)
    dst3 = dst_p.reshape(_NT, _NCH, _CH)
    src2 = src_p.reshape(_EPAD // _CH, _CH)
    dst2 = dst_p.reshape(_EPAD // _CH, _CH)
    ea_p = jnp.pad(edge_attr, ((0, pad), (0, 8 - _EDGE_F)))

    node_in = _H + _STATIC
    w1h = W1[:_H]
    w1s = W1[_H:node_in]
    w1b = jnp.concatenate([W1[node_in:], jnp.zeros((8 - _EDGE_F, _EMB), f32)])
    wla = Wl1[:_EMB]
    wlb = Wl1[_EMB:2 * _EMB]
    wlc = jnp.concatenate([Wl1[2 * _EMB:], jnp.zeros((8 - _EDGE_F, _MLP_H), f32)])

    p = _gru(seq_t, node_static, W_ih.T, W_hh.T, b_ih, b_hh,
             w1h, w1s, b1.reshape(1, -1))

    p_pad = jnp.pad(p, ((0, _PPAD - _N), (0, 0)))
    ps = _gather128(p_pad, src2).reshape(_EPAD, _EMB)

    m = _edge_mlp(ps, ea_p, w1b, W2, b2.reshape(1, -1))

    m3 = m.reshape(_NT, _NCH, _CH, _EMB)
    u2 = _scatter_add(m3, dst3, jnp.zeros((_NPAD, _EMB), f32))

    a, b = _ab(u2, wla, wlb, bl1.reshape(1, -1))

    s1 = _gather512(a, src2).reshape(_EPAD, _MLP_H // 2)
    s2 = _gather512(b, dst2).reshape(_EPAD, _MLP_H // 2)

    pred = _pred(s1, s2, ea_p, wlc, Wl2, bl2.reshape(1, 1))
    return pred[:_E]


# speedup vs baseline: 1.7589x; 1.4081x over previous
import functools

import jax
import jax.numpy as jnp
from jax import lax
from jax.experimental import pallas as pl
from jax.experimental.pallas import tpu as pltpu
from jax.experimental.pallas import tpu_sc as plsc

_N = 10000
_E = 320000
_T = 20
_SEQ_IN = 5
_H = 32
_STATIC = 7
_EMB = 128
_EDGE_F = 5
_MLP_H = 512

_NT = 32
_CH = 128
_NCH = 80
_EPAD = _NT * _NCH * _CH
_NPAD = 10112

_mesh = plsc.VectorSubcoreMesh(
    core_axis_name="c", subcore_axis_name="s", num_cores=2, num_subcores=16)


def _gru_body(seq_ref, st_ref, wir, wiz, win, whr, whz, whn,
              bir, biz, bin_, bhr, bhz, bhn, w1h, w1s, b1, out_ref):
    nb = st_ref.shape[0]

    def step(t, h):
        xs = seq_ref[t]
        i_r = jnp.dot(xs, wir[...], preferred_element_type=jnp.float32) + bir[...]
        i_z = jnp.dot(xs, wiz[...], preferred_element_type=jnp.float32) + biz[...]
        i_n = jnp.dot(xs, win[...], preferred_element_type=jnp.float32) + bin_[...]
        h_r = jnp.dot(h, whr[...], preferred_element_type=jnp.float32) + bhr[...]
        h_z = jnp.dot(h, whz[...], preferred_element_type=jnp.float32) + bhz[...]
        h_n = jnp.dot(h, whn[...], preferred_element_type=jnp.float32) + bhn[...]
        r = jax.nn.sigmoid(i_r + h_r)
        z = jax.nn.sigmoid(i_z + h_z)
        n = jnp.tanh(i_n + r * h_n)
        return (1.0 - z) * n + z * h

    h = lax.fori_loop(0, _T, step, jnp.zeros((nb, _H), jnp.float32))
    p = jnp.dot(h, w1h[...], preferred_element_type=jnp.float32)
    p = p + jnp.dot(st_ref[...], w1s[...], preferred_element_type=jnp.float32)
    out_ref[...] = p + b1[...]


def _gru(seq_t, node_static, wiT, whT, b_ih, b_hh, w1h, w1s, b1):
    nb = 1000
    grid = _N // nb
    full = lambda r, c: pl.BlockSpec((r, c), lambda i: (0, 0))
    specs = [
        pl.BlockSpec((_T, nb, _SEQ_IN), lambda i: (0, i, 0)),
        pl.BlockSpec((nb, _STATIC), lambda i: (i, 0)),
        full(_SEQ_IN, _H), full(_SEQ_IN, _H), full(_SEQ_IN, _H),
        full(_H, _H), full(_H, _H), full(_H, _H),
        full(1, _H), full(1, _H), full(1, _H),
        full(1, _H), full(1, _H), full(1, _H),
        full(_H, _EMB), full(_STATIC, _EMB), full(1, _EMB),
    ]
    args = [seq_t, node_static,
            wiT[:, :_H], wiT[:, _H:2 * _H], wiT[:, 2 * _H:],
            whT[:, :_H], whT[:, _H:2 * _H], whT[:, 2 * _H:],
            b_ih[:_H].reshape(1, -1), b_ih[_H:2 * _H].reshape(1, -1),
            b_ih[2 * _H:].reshape(1, -1),
            b_hh[:_H].reshape(1, -1), b_hh[_H:2 * _H].reshape(1, -1),
            b_hh[2 * _H:].reshape(1, -1),
            w1h, w1s, b1]
    return pl.pallas_call(
        _gru_body,
        grid=(grid,),
        in_specs=specs,
        out_specs=pl.BlockSpec((nb, _EMB), lambda i: (i, 0)),
        out_shape=jax.ShapeDtypeStruct((_N, _EMB), jnp.float32),
    )(*args)


def _make_sc_gather(row_shape, dtype, ch, n0, n1):
    ncht = _EPAD // ch
    assert 16 * (n0 + n1) == ncht and n0 % 8 == 0 and n1 % 8 == 0
    maxn = max(n0, n1)

    @functools.partial(
        pl.kernel,
        out_type=jax.ShapeDtypeStruct((ncht, ch) + row_shape, dtype),
        mesh=_mesh,
        scratch_types=[
            pltpu.VMEM((maxn, ch), jnp.int32),
            pltpu.VMEM((ch,) + row_shape, dtype),
            pltpu.VMEM((ch,) + row_shape, dtype),
            pltpu.SemaphoreType.DMA,
            pltpu.SemaphoreType.DMA,
            pltpu.SemaphoreType.DMA,
            pltpu.SemaphoreType.DMA,
        ],
    )
    def gather(tbl, idx, out, idxbuf, rb0, rb1, gs0, gs1, ws0, ws1):
        c = lax.axis_index("c")
        s = lax.axis_index("s")

        def run(nc, base):
            pltpu.sync_copy(idx.at[pl.ds(base, nc)], idxbuf.at[pl.ds(0, nc)])
            pltpu.async_copy(tbl.at[idxbuf.at[0]], rb0, gs0)
            pltpu.async_copy(tbl.at[idxbuf.at[1]], rb1, gs1)

            def step(i, carry):
                j = 2 * i
                pltpu.make_async_copy(tbl.at[idxbuf.at[0]], rb0, gs0).wait()
                pltpu.async_copy(rb0, out.at[base + j], ws0)
                pltpu.make_async_copy(tbl.at[idxbuf.at[0]], rb1, gs1).wait()
                pltpu.async_copy(rb1, out.at[base + j + 1], ws1)

                @pl.when(j + 2 < nc)
                def _():
                    pltpu.make_async_copy(rb0, out.at[base + j], ws0).wait()
                    pltpu.async_copy(tbl.at[idxbuf.at[j + 2]], rb0, gs0)
                    pltpu.make_async_copy(rb1, out.at[base + j + 1], ws1).wait()
                    pltpu.async_copy(tbl.at[idxbuf.at[j + 3]], rb1, gs1)

                return carry

            lax.fori_loop(0, nc // 2, step, 0)
            pltpu.make_async_copy(rb0, out.at[base + nc - 2], ws0).wait()
            pltpu.make_async_copy(rb1, out.at[base + nc - 1], ws1).wait()

        @pl.when(c == 0)
        def _():
            run(n0, s * n0)

        @pl.when(c == 1)
        def _():
            run(n1, 16 * n0 + s * n1)

    return gather


_PPAD = 10112


@functools.partial(
    pl.kernel,
    out_type=jax.ShapeDtypeStruct((_EPAD // _CH, _CH, _MLP_H // 2),
                                  jnp.float32),
    mesh=_mesh,
    scratch_types=[
        pltpu.VMEM((_EPAD // _CH // 32, _CH), jnp.int32),
        pltpu.VMEM((_CH, 128), jnp.float32),
        pltpu.VMEM((_CH, 128), jnp.float32),
        pltpu.VMEM_SHARED((_PPAD, 128), jnp.float32),
        pltpu.SemaphoreType.DMA,
        pltpu.SemaphoreType.DMA,
        pltpu.SemaphoreType.DMA,
        pltpu.SemaphoreType.DMA,
    ],
)
def _gather512(tbl, idx, out, idxbuf, rb0, rb1, stbl, gs0, gs1, ws0, ws1):
    c = lax.axis_index("c")
    s = lax.axis_index("s")
    rpt = _PPAD // 16
    pltpu.sync_copy(tbl.at[pl.ds(s * rpt, rpt), pl.ds(c * 128, 128)],
                    stbl.at[pl.ds(s * rpt, rpt)])
    plsc.subcore_barrier()
    npt = _EPAD // _CH // 16
    nph = npt // 2

    for ph in range(2):
        base = s * npt + ph * nph
        pltpu.sync_copy(idx.at[pl.ds(base, nph)], idxbuf)
        pltpu.async_copy(stbl.at[idxbuf.at[0]], rb0, gs0)
        pltpu.async_copy(stbl.at[idxbuf.at[1]], rb1, gs1)

        def step(i, carry):
            j = 2 * i
            pltpu.make_async_copy(stbl.at[idxbuf.at[0]], rb0, gs0).wait()
            pltpu.async_copy(rb0, out.at[base + j, :, pl.ds(c * 128, 128)],
                             ws0)
            pltpu.make_async_copy(stbl.at[idxbuf.at[0]], rb1, gs1).wait()
            pltpu.async_copy(rb1, out.at[base + j + 1, :,
                                         pl.ds(c * 128, 128)], ws1)

            @pl.when(j + 2 < nph)
            def _():
                pltpu.make_async_copy(
                    rb0, out.at[base + j, :, pl.ds(c * 128, 128)], ws0).wait()
                pltpu.async_copy(stbl.at[idxbuf.at[j + 2]], rb0, gs0)
                pltpu.make_async_copy(
                    rb1, out.at[base + j + 1, :, pl.ds(c * 128, 128)],
                    ws1).wait()
                pltpu.async_copy(stbl.at[idxbuf.at[j + 3]], rb1, gs1)

            return carry

        lax.fori_loop(0, nph // 2, step, 0)
        pltpu.make_async_copy(rb0, out.at[base + nph - 2, :,
                                          pl.ds(c * 128, 128)], ws0).wait()
        pltpu.make_async_copy(rb1, out.at[base + nph - 1, :,
                                          pl.ds(c * 128, 128)], ws1).wait()


@functools.partial(
    pl.kernel,
    out_type=jax.ShapeDtypeStruct((_EPAD // _CH, _CH, _EMB), jnp.float32),
    mesh=_mesh,
    scratch_types=[
        pltpu.VMEM((_NCH, _CH), jnp.int32),
        pltpu.VMEM((_CH, _EMB), jnp.float32),
        pltpu.VMEM((_CH, _EMB), jnp.float32),
        pltpu.VMEM_SHARED((_PPAD, _EMB), jnp.float32),
        pltpu.SemaphoreType.DMA,
        pltpu.SemaphoreType.DMA,
        pltpu.SemaphoreType.DMA,
        pltpu.SemaphoreType.DMA,
    ],
)
def _gather128(tbl, idx, out, idxbuf, rb0, rb1, ptbl, gs0, gs1, ws0, ws1):
    c = lax.axis_index("c")
    s = lax.axis_index("s")
    rpt = _PPAD // 16
    pltpu.sync_copy(tbl.at[pl.ds(s * rpt, rpt)], ptbl.at[pl.ds(s * rpt, rpt)])
    plsc.subcore_barrier()
    w = s * 2 + c
    base = w * _NCH
    pltpu.sync_copy(idx.at[pl.ds(base, _NCH)], idxbuf)
    pltpu.async_copy(ptbl.at[idxbuf.at[0]], rb0, gs0)
    pltpu.async_copy(ptbl.at[idxbuf.at[1]], rb1, gs1)

    def step(i, carry):
        j = 2 * i
        pltpu.make_async_copy(ptbl.at[idxbuf.at[0]], rb0, gs0).wait()
        pltpu.async_copy(rb0, out.at[base + j], ws0)
        pltpu.make_async_copy(ptbl.at[idxbuf.at[0]], rb1, gs1).wait()
        pltpu.async_copy(rb1, out.at[base + j + 1], ws1)

        @pl.when(j + 2 < _NCH)
        def _():
            pltpu.make_async_copy(rb0, out.at[base + j], ws0).wait()
            pltpu.async_copy(ptbl.at[idxbuf.at[j + 2]], rb0, gs0)
            pltpu.make_async_copy(rb1, out.at[base + j + 1], ws1).wait()
            pltpu.async_copy(ptbl.at[idxbuf.at[j + 3]], rb1, gs1)

        return carry

    lax.fori_loop(0, _NCH // 2, step, 0)
    pltpu.make_async_copy(rb0, out.at[base + _NCH - 2], ws0).wait()
    pltpu.make_async_copy(rb1, out.at[base + _NCH - 1], ws1).wait()


@functools.partial(
    pl.kernel,
    out_type=jax.ShapeDtypeStruct((2, _NPAD, _EMB), jnp.float32),
    mesh=_mesh,
    scratch_types=[
        pltpu.VMEM((_CH, _EMB), jnp.float32),
        pltpu.VMEM((_NCH, _CH), jnp.int32),
        pltpu.VMEM_SHARED((_NPAD, _EMB), jnp.float32),
    ],
)
def _scatter_add(m3, idx3, zeros, out, mbuf, idxbuf, acc):
    c = lax.axis_index("c")
    s = lax.axis_index("s")
    w = s * 2 + c
    rpt = _NPAD // 16
    pltpu.sync_copy(zeros.at[pl.ds(s * rpt, rpt)], acc.at[pl.ds(s * rpt, rpt)])
    plsc.subcore_barrier()
    pltpu.sync_copy(idx3.at[w], idxbuf)

    def step(j, carry):
        pltpu.sync_copy(m3.at[w, j], mbuf)
        pltpu.sync_copy(mbuf, acc.at[idxbuf.at[j]], add=True)
        return carry

    lax.fori_loop(0, _NCH, step, 0)
    plsc.subcore_barrier()
    pltpu.sync_copy(acc.at[pl.ds(s * rpt, rpt)], out.at[c, pl.ds(s * rpt, rpt)])


def _mlp_body(ps_ref, ea_ref, w1b, w2, b2, out_ref):
    t = ps_ref[...] + jnp.dot(ea_ref[...], w1b[...],
                              preferred_element_type=jnp.float32)
    t = jax.nn.relu(t)
    out_ref[...] = jnp.dot(t, w2[...], preferred_element_type=jnp.float32) + b2[...]


def _edge_mlp(ps, ea, w1b, w2, b2):
    blk = 2048
    grid = _EPAD // blk
    full = lambda r, c: pl.BlockSpec((r, c), lambda i: (0, 0))
    return pl.pallas_call(
        _mlp_body,
        grid=(grid,),
        in_specs=[
            pl.BlockSpec((blk, _EMB), lambda i: (i, 0)),
            pl.BlockSpec((blk, 8), lambda i: (i, 0)),
            full(8, _EMB), full(_EMB, _EMB), full(1, _EMB),
        ],
        out_specs=pl.BlockSpec((blk, _EMB), lambda i: (i, 0)),
        out_shape=jax.ShapeDtypeStruct((_EPAD, _EMB), jnp.float32),
    )(ps, ea, w1b, w2, b2)


def _rne_bf16_bits(x):
    u = lax.bitcast_convert_type(x, jnp.uint32)
    return (u + jnp.uint32(0x7FFF) + ((u >> 16) & jnp.uint32(1))) >> 16


def _pack_bf16(x):
    half = x.shape[1] // 2
    lo = _rne_bf16_bits(x[:, :half])
    hi = _rne_bf16_bits(x[:, half:])
    return lax.bitcast_convert_type(lo | (hi << 16), jnp.float32)


def _unpack_bf16(x):
    u = lax.bitcast_convert_type(x, jnp.uint32)
    lo = lax.bitcast_convert_type(u << 16, jnp.float32)
    hi = lax.bitcast_convert_type(u & jnp.uint32(0xFFFF0000), jnp.float32)
    return lo, hi


def _ab_body(u2_ref, wla, wlb, bl1, a_ref, b_ref):
    u = u2_ref[0] + u2_ref[1]
    a = jnp.dot(u, wla[...], preferred_element_type=jnp.float32) + bl1[...]
    b = jnp.dot(u, wlb[...], preferred_element_type=jnp.float32)
    a_ref[...] = _pack_bf16(a)
    b_ref[...] = _pack_bf16(b)


def _ab(u2, wla, wlb, bl1):
    nb = 1000
    grid = _N // nb
    full = lambda r, c: pl.BlockSpec((r, c), lambda i: (0, 0))
    return pl.pallas_call(
        _ab_body,
        grid=(grid,),
        in_specs=[
            pl.BlockSpec((2, nb, _EMB), lambda i: (0, i, 0)),
            full(_EMB, _MLP_H), full(_EMB, _MLP_H), full(1, _MLP_H),
        ],
        out_specs=[
            pl.BlockSpec((nb, _MLP_H // 2), lambda i: (i, 0)),
            pl.BlockSpec((nb, _MLP_H // 2), lambda i: (i, 0)),
        ],
        out_shape=[
            jax.ShapeDtypeStruct((_N, _MLP_H // 2), jnp.float32),
            jax.ShapeDtypeStruct((_N, _MLP_H // 2), jnp.float32),
        ],
    )(u2, wla, wlb, bl1)


def _pred_body(s1_ref, s2_ref, ea_ref, wlc_lo, wlc_hi, wl2_lo, wl2_hi,
               bl2, out_ref):
    s1_lo, s1_hi = _unpack_bf16(s1_ref[...])
    s2_lo, s2_hi = _unpack_bf16(s2_ref[...])
    ea = ea_ref[...]
    h_lo = s1_lo + s2_lo + jnp.dot(ea, wlc_lo[...],
                                   preferred_element_type=jnp.float32)
    h_hi = s1_hi + s2_hi + jnp.dot(ea, wlc_hi[...],
                                   preferred_element_type=jnp.float32)
    h_lo = jax.nn.relu(h_lo)
    h_hi = jax.nn.relu(h_hi)
    logit = (jnp.dot(h_lo, wl2_lo[...], preferred_element_type=jnp.float32)
             + jnp.dot(h_hi, wl2_hi[...], preferred_element_type=jnp.float32)
             + bl2[...])
    out_ref[...] = jax.nn.sigmoid(logit)


def _pred(s1, s2, ea, wlc, wl2, bl2):
    blk = 1024
    grid = _EPAD // blk
    hh = _MLP_H // 2
    full = lambda r, c: pl.BlockSpec((r, c), lambda i: (0, 0))
    return pl.pallas_call(
        _pred_body,
        grid=(grid,),
        in_specs=[
            pl.BlockSpec((blk, hh), lambda i: (i, 0)),
            pl.BlockSpec((blk, hh), lambda i: (i, 0)),
            pl.BlockSpec((blk, 8), lambda i: (i, 0)),
            full(8, hh), full(8, hh), full(hh, 1), full(hh, 1), full(1, 1),
        ],
        out_specs=pl.BlockSpec((blk, 1), lambda i: (i, 0)),
        out_shape=jax.ShapeDtypeStruct((_EPAD, 1), jnp.float32),
    )(s1, s2, ea, wlc[:, :hh], wlc[:, hh:], wl2[:hh], wl2[hh:], bl2)


def kernel(player_seq, node_static, edge_index, edge_attr,
           W_ih, W_hh, b_ih, b_hh, W1, b1, W2, b2, Wl1, bl1, Wl2, bl2):
    f32 = jnp.float32
    pad = _EPAD - _E

    seq_t = jnp.transpose(player_seq, (1, 0, 2))
    src_p = jnp.concatenate([edge_index[0], jnp.zeros((pad,), jnp.int32)])
    dst_p = jnp.concatenate([edge_index[1], jnp.full((pad,), _N, jnp.int32)])
    src3 = src_p.reshape(_NT, _NCH, _CH)
    dst3 = dst_p.reshape(_NT, _NCH, _CH)
    src2 = src_p.reshape(_EPAD // _CH, _CH)
    dst2 = dst_p.reshape(_EPAD // _CH, _CH)
    ea_p = jnp.pad(edge_attr, ((0, pad), (0, 8 - _EDGE_F)))

    node_in = _H + _STATIC
    w1h = W1[:_H]
    w1s = W1[_H:node_in]
    w1b = jnp.concatenate([W1[node_in:], jnp.zeros((8 - _EDGE_F, _EMB), f32)])
    wla = Wl1[:_EMB]
    wlb = Wl1[_EMB:2 * _EMB]
    wlc = jnp.concatenate([Wl1[2 * _EMB:], jnp.zeros((8 - _EDGE_F, _MLP_H), f32)])

    p = _gru(seq_t, node_static, W_ih.T, W_hh.T, b_ih, b_hh,
             w1h, w1s, b1.reshape(1, -1))

    p_pad = jnp.pad(p, ((0, _PPAD - _N), (0, 0)))
    ps = _gather128(p_pad, src2).reshape(_EPAD, _EMB)

    m = _edge_mlp(ps, ea_p, w1b, W2, b2.reshape(1, -1))

    m3 = m.reshape(_NT, _NCH, _CH, _EMB)
    u2 = _scatter_add(m3, dst3, jnp.zeros((_NPAD, _EMB), f32))

    a, b = _ab(u2, wla, wlb, bl1.reshape(1, -1))

    a_pad = jnp.pad(a, ((0, _PPAD - _N), (0, 0)))
    b_pad = jnp.pad(b, ((0, _PPAD - _N), (0, 0)))
    s1 = _gather512(a_pad, src2).reshape(_EPAD, _MLP_H // 2)
    s2 = _gather512(b_pad, dst2).reshape(_EPAD, _MLP_H // 2)

    pred = _pred(s1, s2, ea_p, wlc, Wl2, bl2.reshape(1, 1))
    return pred[:_E]
